# baseline (speedup 1.0000x reference)
import jax
import jax.numpy as jnp
from jax import lax
from jax.experimental import pallas as pl
from jax.experimental.pallas import tpu as pltpu
from jax.experimental.pallas import tpu_sc as plsc

_CX = 10

_NC = 2
_NS = 16
_NW = _NC * _NS


def _x_copy_body(x_hbm, x_out, x_v, xin_sem, xout_sem):
    nx = x_hbm.shape[0] // _CX
    loads = []
    for i in range(_CX):
        c = pltpu.make_async_copy(
            x_hbm.at[pl.ds(i * nx, nx), :], x_v.at[pl.ds(i * nx, nx), :],
            xin_sem.at[i])
        c.start()
        loads.append(c)
    stores = []
    for i in range(_CX):
        loads[i].wait()
        s = pltpu.make_async_copy(
            x_v.at[pl.ds(i * nx, nx), :], x_out.at[pl.ds(i * nx, nx), :],
            xout_sem.at[i])
        s.start()
        stores.append(s)
    for s in stores:
        s.wait()


_EC = 25


def _sc_e_copy(e_hbm, e_out, b0, b1, s0, s1, t0, t1):
    wid = lax.axis_index("s") * _NC + lax.axis_index("c")
    er = e_hbm.shape[0] // _NW
    nc = er // _EC
    base = wid * er
    bufs = (b0, b1)
    isems = (s0, s1)
    osems = (t0, t1)
    loads = {}
    stores = {}
    for i in range(2):
        c = pltpu.make_async_copy(
            e_hbm.at[pl.ds(base + i * nc, nc)], bufs[i], isems[i])
        c.start()
        loads[i] = c
    for i in range(_EC):
        loads[i].wait()
        s = pltpu.make_async_copy(
            bufs[i % 2], e_out.at[pl.ds(base + i * nc, nc)], osems[i % 2])
        s.start()
        stores[i] = s
        nxt = i + 2
        if nxt < _EC:
            stores[i].wait()
            c = pltpu.make_async_copy(
                e_hbm.at[pl.ds(base + nxt * nc, nc)], bufs[nxt % 2],
                isems[nxt % 2])
            c.start()
            loads[nxt] = c
    for i in range(_EC - 2, _EC):
        stores[i].wait()


def kernel(x, x_lstm, encoded_z_gnss, edge_index, edge_attr):
    n_nodes, d_feat = x.shape
    n_edges, d_edge = edge_attr.shape
    x_out = pl.pallas_call(
        _x_copy_body,
        out_shape=jax.ShapeDtypeStruct(x.shape, x.dtype),
        in_specs=[pl.BlockSpec(memory_space=pl.ANY)],
        out_specs=pl.BlockSpec(memory_space=pl.ANY),
        scratch_shapes=[
            pltpu.MemorySpace.VMEM((n_nodes, d_feat), jnp.float32),
            pltpu.SemaphoreType.DMA((_CX,)),
            pltpu.SemaphoreType.DMA((_CX,)),
        ],
    )(x)
    e_copy = pl.kernel(
        _sc_e_copy,
        out_type=jax.ShapeDtypeStruct(edge_attr.shape, edge_attr.dtype),
        mesh=plsc.VectorSubcoreMesh(
            core_axis_name="c", subcore_axis_name="s",
            num_cores=_NC, num_subcores=_NS,
        ),
        scratch_types=[
            pltpu.MemorySpace.VMEM((n_edges // _NW // _EC, d_edge), jnp.float32),
            pltpu.MemorySpace.VMEM((n_edges // _NW // _EC, d_edge), jnp.float32),
            pltpu.SemaphoreType.DMA,
            pltpu.SemaphoreType.DMA,
            pltpu.SemaphoreType.DMA,
            pltpu.SemaphoreType.DMA,
        ],
    )
    e_out = e_copy(edge_attr)
    return (x_out, e_out)


# speedup vs baseline: 1.1565x; 1.1565x over previous
import jax
import jax.numpy as jnp
from jax.experimental import pallas as pl
from jax.experimental.pallas import tpu as pltpu

_GRID = 10


def _copy_body(x_ref, e_ref, x_out, e_out):
    x_out[...] = x_ref[...]
    e_out[...] = e_ref[...]


def kernel(x, x_lstm, encoded_z_gnss, edge_index, edge_attr):
    n_nodes, d_feat = x.shape
    n_edges, d_edge = edge_attr.shape
    bx = n_nodes // _GRID
    be = n_edges // _GRID
    x_out, e_out = pl.pallas_call(
        _copy_body,
        grid=(_GRID,),
        out_shape=(
            jax.ShapeDtypeStruct(x.shape, x.dtype),
            jax.ShapeDtypeStruct(edge_attr.shape, edge_attr.dtype),
        ),
        in_specs=[
            pl.BlockSpec((bx, d_feat), lambda i: (i, 0)),
            pl.BlockSpec((be, d_edge), lambda i: (i, 0)),
        ],
        out_specs=(
            pl.BlockSpec((bx, d_feat), lambda i: (i, 0)),
            pl.BlockSpec((be, d_edge), lambda i: (i, 0)),
        ),
        compiler_params=pltpu.CompilerParams(
            dimension_semantics=("arbitrary",),
            allow_input_fusion=[False, True],
        ),
    )(x, edge_attr)
    return (x_out, e_out)
